# baseline (speedup 1.0000x reference)
import functools

import jax
import jax.numpy as jnp
from jax import lax
from jax.experimental import pallas as pl
from jax.experimental.pallas import tpu as pltpu

_DELTA_V = 0.5
_DELTA_D = 3.0
_NL = 4


def _pick_chunk(hw):
    for c in (8192, 4096, 2048, 1024, 512, 256, 128):
        if hw % c == 0:
            return c
    return hw


def _pass1_body(emb_ref, lab_ref, stats_ref, *, nchunks):
    b = pl.program_id(0)
    j = pl.program_id(1)

    @pl.when(j == 0)
    def _():
        stats_ref[...] = jnp.zeros_like(stats_ref)

    emb = emb_ref[0]
    lab = lab_ref[0, 0, :]
    chunk = emb.shape[1]
    lane_ids = lax.broadcasted_iota(jnp.int32, (_NL, chunk), 0) + 1
    masks = (lab[None, :] == lane_ids).astype(jnp.float32)
    sums = lax.dot_general(masks, emb, (((1,), (1,)), ((), ())),
                           preferred_element_type=jnp.float32)
    counts = jnp.sum(masks, axis=1, keepdims=True)
    c = emb.shape[0]
    pad_cols = jnp.zeros((_NL, 128 - c - 1), dtype=jnp.float32)
    row_block = jnp.concatenate([sums, counts, pad_cols], axis=1)
    pad_rows = jnp.zeros((8 - _NL, 128), dtype=jnp.float32)
    stats_ref[0] += jnp.concatenate([row_block, pad_rows], axis=0)


def _pass2_body(emb_ref, lab_ref, stats_ref, var_ref, dist_ref,
                acc_ref, accb_ref, *, nb, nchunks):
    b = pl.program_id(0)
    j = pl.program_id(1)

    @pl.when(jnp.logical_and(b == 0, j == 0))
    def _():
        acc_ref[...] = jnp.zeros_like(acc_ref)

    @pl.when(j == 0)
    def _():
        accb_ref[...] = jnp.zeros_like(accb_ref)

    emb = emb_ref[0]
    lab = lab_ref[0, 0, :]
    c = emb.shape[0]
    chunk = emb.shape[1]

    stats_b = stats_ref[b]
    cnt = stats_b[0:_NL, c:c + 1]
    safe_cnt = jnp.where(cnt > 0, cnt, 1.0)
    mu = stats_b[0:_NL, 0:c] / safe_cnt

    dots = lax.dot_general(mu, emb, (((1,), (0,)), ((), ())),
                           preferred_element_type=jnp.float32)
    sq_e = jnp.sum(emb * emb, axis=0, keepdims=True)
    sq_mu = jnp.sum(mu * mu, axis=1, keepdims=True)
    d2 = jnp.maximum(sq_e - 2.0 * dots + sq_mu, 0.0)
    d = jnp.sqrt(d2)
    lane_ids = lax.broadcasted_iota(jnp.int32, (_NL, chunk), 0) + 1
    masks = (lab[None, :] == lane_ids).astype(jnp.float32)
    hinge = jnp.maximum(d - _DELTA_V, 0.0)
    lane_sums = jnp.sum(hinge * hinge * masks, axis=1, keepdims=True)
    pad_cols = jnp.zeros((_NL, 127), dtype=jnp.float32)
    pad_rows = jnp.zeros((8 - _NL, 128), dtype=jnp.float32)
    accb_ref[...] += jnp.concatenate(
        [jnp.concatenate([lane_sums, pad_cols], axis=1), pad_rows], axis=0)

    @pl.when(j == nchunks - 1)
    def _():
        acc_ref[pl.ds(b * _NL, _NL), :] = accb_ref[0:_NL, :]

    @pl.when(jnp.logical_and(b == nb - 1, j == nchunks - 1))
    def _():
        var_loss = jnp.float32(0.0)
        dist_loss = jnp.float32(0.0)
        for bb in range(nb):
            stats_bb = stats_ref[bb]
            cnt_b = stats_bb[0:_NL, c:c + 1]
            has = cnt_b > 0
            safe = jnp.where(has, cnt_b, 1.0)
            varsums = acc_ref[pl.ds(bb * _NL, _NL), 0:1]
            batch_var = jnp.sum(jnp.where(has, varsums / safe, 0.0))
            nl = jnp.sum(has.astype(jnp.float32))
            mu_b = jnp.where(has, stats_bb[0:_NL, 0:c] / safe, 0.0)
            contrib = jnp.float32(0.0)
            for i in range(_NL):
                for k in range(i + 1, _NL):
                    diff = mu_b[i:i + 1, :] - mu_b[k:k + 1, :]
                    pd2 = jnp.sum(diff * diff)
                    pd = jnp.where(pd2 > 0,
                                   jnp.sqrt(jnp.where(pd2 > 0, pd2, 1.0)),
                                   0.0)
                    both = (cnt_b[i, 0] * cnt_b[k, 0]) > 0
                    h = jnp.maximum(_DELTA_D - pd, 0.0)
                    contrib += 2.0 * jnp.where(both, h * h, 0.0)
            new_var = (var_loss + batch_var) / nl
            var_loss = jnp.where(nl > 0, new_var, var_loss)
            new_dist = (dist_loss + jnp.where(nl > 1, contrib, 0.0)) / (
                2.0 * nl * (nl - 1.0))
            dist_loss = jnp.where(nl > 0, new_dist, dist_loss)
        var_ref[...] = jnp.reshape(var_loss / nb, (1, 1))
        dist_ref[...] = jnp.reshape(dist_loss / nb, (1, 1))


def _run(emb3, lab3, interpret=False):
    nb, c, hw = emb3.shape
    chunk = _pick_chunk(hw)
    nchunks = hw // chunk
    grid = (nb, nchunks)

    emb_spec = pl.BlockSpec((1, c, chunk), lambda b, j: (b, 0, j))
    lab_spec = pl.BlockSpec((1, 1, chunk), lambda b, j: (b, 0, j))

    stats = pl.pallas_call(
        functools.partial(_pass1_body, nchunks=nchunks),
        grid=grid,
        in_specs=[emb_spec, lab_spec],
        out_specs=pl.BlockSpec((1, 8, 128), lambda b, j: (b, 0, 0)),
        out_shape=jax.ShapeDtypeStruct((nb, 8, 128), jnp.float32),
        compiler_params=pltpu.CompilerParams(
            dimension_semantics=("arbitrary", "arbitrary")),
        interpret=interpret,
    )(emb3, lab3)

    var, dist = pl.pallas_call(
        functools.partial(_pass2_body, nb=nb, nchunks=nchunks),
        grid=grid,
        in_specs=[emb_spec, lab_spec,
                  pl.BlockSpec((nb, 8, 128), lambda b, j: (0, 0, 0))],
        out_specs=[pl.BlockSpec((1, 1), lambda b, j: (0, 0)),
                   pl.BlockSpec((1, 1), lambda b, j: (0, 0))],
        out_shape=[jax.ShapeDtypeStruct((1, 1), jnp.float32),
                   jax.ShapeDtypeStruct((1, 1), jnp.float32)],
        scratch_shapes=[pltpu.VMEM((8 * nb, 128), jnp.float32),
                        pltpu.VMEM((8, 128), jnp.float32)],
        compiler_params=pltpu.CompilerParams(
            dimension_semantics=("arbitrary", "arbitrary")),
        interpret=interpret,
    )(emb3, lab3, stats)

    return var[0, 0], dist[0, 0]


def kernel(embedding_tensor, instance_labels):
    nb, c, h, w = embedding_tensor.shape
    emb3 = embedding_tensor.reshape(nb, c, h * w)
    lab3 = instance_labels.reshape(nb, 1, h * w).astype(jnp.int32)
    return _run(emb3, lab3)
